# baseline (speedup 1.0000x reference)
import functools

import jax
import jax.numpy as jnp
from jax import lax
from jax.experimental import pallas as pl
from jax.experimental.pallas import tpu as pltpu
from jax.experimental.pallas import tpu_sc as plsc

_LANE = 128
_NC = 2
_NS = 16
_NW = _NC * _NS


def _make_kernel(v, d, b, h):
    nbt = b // _LANE
    assert nbt == _NW and d == 32
    njt = d // 8
    mesh = plsc.VectorSubcoreMesh(core_axis_name="c", subcore_axis_name="s")

    @functools.partial(
        pl.kernel,
        mesh=mesh,
        out_type=jax.ShapeDtypeStruct((h, njt, nbt, 8 * _LANE), jnp.float32),
        scratch_types=[
            pltpu.VMEM((h, _LANE), jnp.int32),
            pltpu.VMEM((_LANE, d), jnp.float32),
            pltpu.VMEM((_LANE, d), jnp.float32),
            pltpu.VMEM((d * _LANE,), jnp.float32),
            pltpu.VMEM((d * _LANE,), jnp.float32),
            pltpu.SemaphoreType.DMA,
            pltpu.SemaphoreType.DMA,
            pltpu.SemaphoreType.DMA,
            pltpu.SemaphoreType.DMA,
        ],
        compiler_params=pltpu.CompilerParams(
            use_tc_tiling_on_sc=False, needs_layout_passes=False),
    )
    def k(table_hbm, sent_t_hbm, out_hbm, idx_v, ga, gb, ta, tb,
          gsa, gsb, osa, osb):
        wid = lax.axis_index("s") * _NC + lax.axis_index("c")
        pltpu.sync_copy(sent_t_hbm.at[:, pl.ds(wid * _LANE, _LANE)], idx_v)

        iota = lax.iota(jnp.int32, 16)
        jlo = iota * _LANE
        jhi = jlo + 16 * _LANE

        def fire_gather(hh, gbuf, gsem):
            pltpu.async_copy(table_hbm.at[idx_v.at[hh]], gbuf, gsem)

        def wait_gather(gbuf, gsem):
            pltpu.make_async_copy(table_hbm.at[pl.ds(0, _LANE)], gbuf, gsem).wait()

        def transpose(gbuf, tbuf):
            def cbody(c, carry):
                vlo = gbuf[c, pl.ds(0, 16)]
                vhi = gbuf[c, pl.ds(16, 16)]
                plsc.store_scatter(tbuf, [jlo + c], vlo)
                plsc.store_scatter(tbuf, [jhi + c], vhi)
                return carry
            lax.fori_loop(0, _LANE, cbody, 0)

        def fire_out(hh, tbuf, osem):
            for jt in range(njt):
                pltpu.async_copy(
                    tbuf.at[pl.ds(jt * 8 * _LANE, 8 * _LANE)],
                    out_hbm.at[hh, jt, wid],
                    osem,
                )

        def wait_out(tbuf, osem):
            for jt in range(njt):
                pltpu.make_async_copy(
                    tbuf.at[pl.ds(jt * 8 * _LANE, 8 * _LANE)],
                    out_hbm.at[0, 0, 0],
                    osem,
                ).wait()

        fire_gather(0, ga, gsa)

        fire_gather(1, gb, gsb)
        wait_gather(ga, gsa)
        transpose(ga, ta)
        fire_out(0, ta, osa)
        fire_gather(2, ga, gsa)
        wait_gather(gb, gsb)
        transpose(gb, tb)
        fire_out(1, tb, osb)

        def body(g, carry):
            h0 = 2 * g
            fire_gather(h0 + 1, gb, gsb)
            wait_gather(ga, gsa)
            wait_out(ta, osa)
            transpose(ga, ta)
            fire_out(h0, ta, osa)
            fire_gather(h0 + 2, ga, gsa)
            wait_gather(gb, gsb)
            wait_out(tb, osb)
            transpose(gb, tb)
            fire_out(h0 + 1, tb, osb)
            return carry

        lax.fori_loop(1, h // 2 - 1, body, 0)

        fire_gather(h - 1, gb, gsb)
        wait_gather(ga, gsa)
        wait_out(ta, osa)
        transpose(ga, ta)
        fire_out(h - 2, ta, osa)
        wait_gather(gb, gsb)
        wait_out(tb, osb)
        transpose(gb, tb)
        fire_out(h - 1, tb, osb)

        wait_out(ta, osa)
        wait_out(tb, osb)

    return k


def kernel(sent, table):
    b, h = sent.shape
    v, d = table.shape
    sent_t = sent.T
    out5 = _make_kernel(v, d, b, h)(table, sent_t)
    out5 = out5.reshape(h, d // 8, b // _LANE, 8, _LANE)
    return out5.transpose(2, 4, 0, 1, 3).reshape(b, h, d)
